# baseline (speedup 1.0000x reference)
import functools

import jax
import jax.numpy as jnp
from jax import lax
from jax.experimental import pallas as pl
from jax.experimental.pallas import tpu as pltpu
from jax.experimental.pallas import tpu_sc as plsc

N_NODES = 10000
N_EDGES = 320000
HIDDEN = 128

NC = 2
NS = 16
NW = NC * NS
CHUNK = 80
NCHUNK = 125
EPW = CHUNK * NCHUNK
RPT = 632
RPT_LAST = N_NODES - 15 * RPT
NBUF = 3
LEAD = 2


def _sc_aggregate(x, edges):
    mesh = plsc.VectorSubcoreMesh(core_axis_name="c", subcore_axis_name="s")

    @functools.partial(
        pl.kernel,
        mesh=mesh,
        out_type=jax.ShapeDtypeStruct((NC, N_NODES, HIDDEN), jnp.float32),
        scratch_types=[
            pltpu.VMEM_SHARED((N_NODES, HIDDEN), jnp.float32),
            pltpu.VMEM((EPW,), jnp.int32),
            pltpu.VMEM((EPW,), jnp.int32),
            pltpu.VMEM((NBUF, CHUNK, HIDDEN), jnp.float32),
            pltpu.SemaphoreType.DMA((NBUF,)),
            pltpu.SemaphoreType.DMA((NBUF,)),
            pltpu.SemaphoreType.DMA,
        ],
    )
    def k(x_hbm, e_hbm, out_hbm, acc, src_v, dst_v, rows_v,
          gsem, ssem, isem):
        c = lax.axis_index("c")
        s = lax.axis_index("s")
        wid = s * NC + c
        @pl.when(jnp.logical_and(c == 0, s < NS - 1))
        def _():
            pltpu.async_copy(x_hbm.at[pl.ds(s * RPT, RPT)],
                             acc.at[pl.ds(s * RPT, RPT)], isem)
        @pl.when(jnp.logical_and(c == 0, s == NS - 1))
        def _():
            pltpu.async_copy(x_hbm.at[pl.ds((NS - 1) * RPT, RPT_LAST)],
                             acc.at[pl.ds((NS - 1) * RPT, RPT_LAST)], isem)
        @pl.when(c == 1)
        def _():
            z = rows_v.at[0]
            def zbody(i, carry):
                r = i // (HIDDEN // 16)
                q = i % (HIDDEN // 16)
                z[r, pl.ds(q * 16, 16)] = jnp.zeros((16,), jnp.float32)
                return carry
            lax.fori_loop(0, CHUNK * (HIDDEN // 16), zbody, 0)
            @pl.when(s < NS - 1)
            def _():
                for kk in range(RPT // CHUNK):
                    pltpu.sync_copy(z, acc.at[pl.ds(s * RPT + kk * CHUNK,
                                                    CHUNK)])
                pltpu.sync_copy(
                    z.at[pl.ds(0, RPT % CHUNK)],
                    acc.at[pl.ds(s * RPT + (RPT // CHUNK) * CHUNK,
                                 RPT % CHUNK)])
            @pl.when(s == NS - 1)
            def _():
                for kk in range(RPT_LAST // CHUNK):
                    pltpu.sync_copy(
                        z, acc.at[pl.ds((NS - 1) * RPT + kk * CHUNK, CHUNK)])
                pltpu.sync_copy(
                    z.at[pl.ds(0, RPT_LAST % CHUNK)],
                    acc.at[pl.ds((NS - 1) * RPT + (RPT_LAST // CHUNK) * CHUNK,
                                 RPT_LAST % CHUNK)])
        pltpu.sync_copy(e_hbm.at[pl.ds(wid * EPW, EPW)], src_v)
        pltpu.sync_copy(e_hbm.at[pl.ds(N_EDGES + wid * EPW, EPW)], dst_v)

        GSUB = 2
        GPART = CHUNK // GSUB

        def start_gather(j, b):
            for i in range(GSUB):
                pltpu.async_copy(
                    x_hbm.at[src_v.at[pl.ds(j * CHUNK + i * GPART, GPART)]],
                    rows_v.at[b].at[pl.ds(i * GPART, GPART)], gsem.at[b])

        def wait_gather(j, b):
            for i in range(GSUB):
                pltpu.make_async_copy(
                    x_hbm.at[src_v.at[pl.ds(j * CHUNK + i * GPART, GPART)]],
                    rows_v.at[b].at[pl.ds(i * GPART, GPART)], gsem.at[b]).wait()

        def start_scatter(j, b):
            for i in range(CHUNK // 16):
                idx = dst_v[pl.ds(j * CHUNK + i * 16, 16)]
                pltpu.async_copy(rows_v.at[b].at[pl.ds(i * 16, 16)],
                                 acc.at[idx], ssem.at[b], add=True)

        def wait_scatter(j, b):
            for i in range(CHUNK // 16):
                idx = dst_v[pl.ds(j * CHUNK + i * 16, 16)]
                pltpu.make_async_copy(rows_v.at[b].at[pl.ds(i * 16, 16)],
                                      acc.at[idx], ssem.at[b]).wait()

        for b in range(LEAD + 1):
            start_gather(b, b)
        @pl.when(jnp.logical_and(c == 0, s < NS - 1))
        def _():
            pltpu.make_async_copy(x_hbm.at[pl.ds(s * RPT, RPT)],
                                  acc.at[pl.ds(s * RPT, RPT)], isem).wait()
        @pl.when(jnp.logical_and(c == 0, s == NS - 1))
        def _():
            pltpu.make_async_copy(
                x_hbm.at[pl.ds((NS - 1) * RPT, RPT_LAST)],
                acc.at[pl.ds((NS - 1) * RPT, RPT_LAST)], isem).wait()
        plsc.subcore_barrier()
        wait_gather(0, 0)
        start_scatter(0, 0)

        def body(g, carry):
            for b in range(NBUF):
                j = g * NBUF + b
                wait_scatter(j - NBUF, b)
                start_gather(j, b)
                jq = j - LEAD
                bq = (b + NBUF - LEAD) % NBUF
                wait_gather(jq, bq)
                start_scatter(jq, bq)
            return carry

        G = (NCHUNK - LEAD) // NBUF
        lax.fori_loop(1, G, body, 0)
        for j in range(G * NBUF, NCHUNK):
            b = j % NBUF
            wait_scatter(j - NBUF, b)
            start_gather(j, b)
            jq = j - LEAD
            wait_gather(jq, jq % NBUF)
            start_scatter(jq, jq % NBUF)
        for jq in range(NCHUNK, NCHUNK + LEAD):
            wait_gather(jq - LEAD, (jq - LEAD) % NBUF)
            start_scatter(jq - LEAD, (jq - LEAD) % NBUF)
        for jt in range(NCHUNK - NBUF, NCHUNK):
            wait_scatter(jt, jt % NBUF)
        plsc.subcore_barrier()
        @pl.when(s < NS - 1)
        def _():
            pltpu.sync_copy(acc.at[pl.ds(s * RPT, RPT)],
                            out_hbm.at[c].at[pl.ds(s * RPT, RPT)])
        @pl.when(s == NS - 1)
        def _():
            pltpu.sync_copy(acc.at[pl.ds((NS - 1) * RPT, RPT_LAST)],
                            out_hbm.at[c].at[pl.ds((NS - 1) * RPT, RPT_LAST)])

    return k(x, edges)


def _tc_dense(partials, W, b, gamma, beta):
    def body(p_ref, w_ref, b_ref, g_ref, be_ref, o_ref):
        h = p_ref[0] + p_ref[1]
        h = jnp.dot(h, w_ref[...], preferred_element_type=jnp.float32)
        h = h + b_ref[...]
        mean = jnp.mean(h, axis=0, keepdims=True)
        var = jnp.mean((h - mean) * (h - mean), axis=0, keepdims=True)
        h = (h - mean) * lax.rsqrt(var + 1e-5) * g_ref[...] + be_ref[...]
        o_ref[...] = jnp.maximum(h, 0.0)

    return pl.pallas_call(
        body,
        out_shape=jax.ShapeDtypeStruct((N_NODES, HIDDEN), jnp.float32),
    )(partials, W, b.reshape(1, HIDDEN), gamma.reshape(1, HIDDEN),
      beta.reshape(1, HIDDEN))


def kernel(x, edge_index, batch, W, b, gamma, beta):
    del batch
    edges = edge_index.astype(jnp.int32).reshape(2 * N_EDGES)
    partials = _sc_aggregate(x, edges)
    return _tc_dense(partials, W, b, gamma, beta)


# speedup vs baseline: 1.0206x; 1.0206x over previous
import functools

import jax
import jax.numpy as jnp
from jax import lax
from jax.experimental import pallas as pl
from jax.experimental.pallas import tpu as pltpu
from jax.experimental.pallas import tpu_sc as plsc

N_NODES = 10000
N_EDGES = 320000
HIDDEN = 128

NC = 2
NS = 16
NW = NC * NS
CHUNK = 80
NCHUNK = 125
EPW = CHUNK * NCHUNK
RPT = 632
RPT_LAST = N_NODES - 15 * RPT
NBUF = 3
LEAD = 2


def _sc_aggregate(x, edges):
    mesh = plsc.VectorSubcoreMesh(core_axis_name="c", subcore_axis_name="s")

    @functools.partial(
        pl.kernel,
        mesh=mesh,
        out_type=jax.ShapeDtypeStruct((NC, N_NODES, HIDDEN), jnp.float32),
        scratch_types=[
            pltpu.VMEM_SHARED((N_NODES, HIDDEN), jnp.float32),
            pltpu.VMEM((EPW,), jnp.int32),
            pltpu.VMEM((EPW,), jnp.int32),
            pltpu.VMEM((NBUF, CHUNK, HIDDEN), jnp.float32),
            pltpu.SemaphoreType.DMA((NBUF,)),
            pltpu.SemaphoreType.DMA((NBUF,)),
            pltpu.SemaphoreType.DMA,
        ],
    )
    def k(x_hbm, e_hbm, out_hbm, acc, src_v, dst_v, rows_v,
          gsem, ssem, isem):
        c = lax.axis_index("c")
        s = lax.axis_index("s")
        wid = s * NC + c
        @pl.when(jnp.logical_and(c == 0, s < NS - 1))
        def _():
            pltpu.async_copy(x_hbm.at[pl.ds(s * RPT, RPT)],
                             acc.at[pl.ds(s * RPT, RPT)], isem)
        @pl.when(jnp.logical_and(c == 0, s == NS - 1))
        def _():
            pltpu.async_copy(x_hbm.at[pl.ds((NS - 1) * RPT, RPT_LAST)],
                             acc.at[pl.ds((NS - 1) * RPT, RPT_LAST)], isem)
        @pl.when(c == 1)
        def _():
            z = rows_v.at[0]
            def zbody(r, carry):
                for q in range(HIDDEN // 16):
                    z[r, pl.ds(q * 16, 16)] = jnp.zeros((16,), jnp.float32)
                return carry
            lax.fori_loop(0, CHUNK, zbody, 0)
            @pl.when(s < NS - 1)
            def _():
                for kk in range(RPT // CHUNK):
                    pltpu.sync_copy(z, acc.at[pl.ds(s * RPT + kk * CHUNK,
                                                    CHUNK)])
                pltpu.sync_copy(
                    z.at[pl.ds(0, RPT % CHUNK)],
                    acc.at[pl.ds(s * RPT + (RPT // CHUNK) * CHUNK,
                                 RPT % CHUNK)])
            @pl.when(s == NS - 1)
            def _():
                for kk in range(RPT_LAST // CHUNK):
                    pltpu.sync_copy(
                        z, acc.at[pl.ds((NS - 1) * RPT + kk * CHUNK, CHUNK)])
                pltpu.sync_copy(
                    z.at[pl.ds(0, RPT_LAST % CHUNK)],
                    acc.at[pl.ds((NS - 1) * RPT + (RPT_LAST // CHUNK) * CHUNK,
                                 RPT_LAST % CHUNK)])
        pltpu.sync_copy(e_hbm.at[pl.ds(wid * EPW, EPW)], src_v)
        pltpu.sync_copy(e_hbm.at[pl.ds(N_EDGES + wid * EPW, EPW)], dst_v)

        GSUB = 2
        GPART = CHUNK // GSUB

        def start_gather(j, b):
            for i in range(GSUB):
                pltpu.async_copy(
                    x_hbm.at[src_v.at[pl.ds(j * CHUNK + i * GPART, GPART)]],
                    rows_v.at[b].at[pl.ds(i * GPART, GPART)], gsem.at[b])

        def wait_gather(j, b):
            for i in range(GSUB):
                pltpu.make_async_copy(
                    x_hbm.at[src_v.at[pl.ds(j * CHUNK + i * GPART, GPART)]],
                    rows_v.at[b].at[pl.ds(i * GPART, GPART)], gsem.at[b]).wait()

        def start_scatter(j, b):
            for i in range(CHUNK // 16):
                idx = dst_v[pl.ds(j * CHUNK + i * 16, 16)]
                pltpu.async_copy(rows_v.at[b].at[pl.ds(i * 16, 16)],
                                 acc.at[idx], ssem.at[b], add=True)

        def wait_scatter(j, b):
            for i in range(CHUNK // 16):
                idx = dst_v[pl.ds(j * CHUNK + i * 16, 16)]
                pltpu.make_async_copy(rows_v.at[b].at[pl.ds(i * 16, 16)],
                                      acc.at[idx], ssem.at[b]).wait()

        for b in range(LEAD + 1):
            start_gather(b, b)
        @pl.when(jnp.logical_and(c == 0, s < NS - 1))
        def _():
            pltpu.make_async_copy(x_hbm.at[pl.ds(s * RPT, RPT)],
                                  acc.at[pl.ds(s * RPT, RPT)], isem).wait()
        @pl.when(jnp.logical_and(c == 0, s == NS - 1))
        def _():
            pltpu.make_async_copy(
                x_hbm.at[pl.ds((NS - 1) * RPT, RPT_LAST)],
                acc.at[pl.ds((NS - 1) * RPT, RPT_LAST)], isem).wait()
        plsc.subcore_barrier()
        wait_gather(0, 0)
        start_scatter(0, 0)

        def body(g, carry):
            for b in range(NBUF):
                j = g * NBUF + b
                wait_scatter(j - NBUF, b)
                start_gather(j, b)
                jq = j - LEAD
                bq = (b + NBUF - LEAD) % NBUF
                wait_gather(jq, bq)
                start_scatter(jq, bq)
            return carry

        G = (NCHUNK - LEAD) // NBUF
        lax.fori_loop(1, G, body, 0)
        for j in range(G * NBUF, NCHUNK):
            b = j % NBUF
            wait_scatter(j - NBUF, b)
            start_gather(j, b)
            jq = j - LEAD
            wait_gather(jq, jq % NBUF)
            start_scatter(jq, jq % NBUF)
        for jq in range(NCHUNK, NCHUNK + LEAD):
            wait_gather(jq - LEAD, (jq - LEAD) % NBUF)
            start_scatter(jq - LEAD, (jq - LEAD) % NBUF)
        for jt in range(NCHUNK - NBUF, NCHUNK):
            wait_scatter(jt, jt % NBUF)
        plsc.subcore_barrier()
        @pl.when(s < NS - 1)
        def _():
            pltpu.sync_copy(acc.at[pl.ds(s * RPT, RPT)],
                            out_hbm.at[c].at[pl.ds(s * RPT, RPT)])
        @pl.when(s == NS - 1)
        def _():
            pltpu.sync_copy(acc.at[pl.ds((NS - 1) * RPT, RPT_LAST)],
                            out_hbm.at[c].at[pl.ds((NS - 1) * RPT, RPT_LAST)])

    return k(x, edges)


def _tc_dense(partials, W, b, gamma, beta):
    def body(p_ref, w_ref, b_ref, g_ref, be_ref, o_ref):
        h = p_ref[0] + p_ref[1]
        h = jnp.dot(h, w_ref[...], preferred_element_type=jnp.float32)
        h = h + b_ref[...]
        mean = jnp.mean(h, axis=0, keepdims=True)
        var = jnp.mean((h - mean) * (h - mean), axis=0, keepdims=True)
        h = (h - mean) * lax.rsqrt(var + 1e-5) * g_ref[...] + be_ref[...]
        o_ref[...] = jnp.maximum(h, 0.0)

    return pl.pallas_call(
        body,
        out_shape=jax.ShapeDtypeStruct((N_NODES, HIDDEN), jnp.float32),
    )(partials, W, b.reshape(1, HIDDEN), gamma.reshape(1, HIDDEN),
      beta.reshape(1, HIDDEN))


def kernel(x, edge_index, batch, W, b, gamma, beta):
    del batch
    edges = edge_index.astype(jnp.int32).reshape(2 * N_EDGES)
    partials = _sc_aggregate(x, edges)
    return _tc_dense(partials, W, b, gamma, beta)


# speedup vs baseline: 1.0292x; 1.0084x over previous
import functools

import jax
import jax.numpy as jnp
from jax import lax
from jax.experimental import pallas as pl
from jax.experimental.pallas import tpu as pltpu
from jax.experimental.pallas import tpu_sc as plsc

N_NODES = 10000
N_EDGES = 320000
HIDDEN = 128

NC = 2
NS = 16
NW = NC * NS
CHUNK = 80
NCHUNK = 125
EPW = CHUNK * NCHUNK
RPT = 632
RPT_LAST = N_NODES - 15 * RPT
NBUF = 3
LEAD = 2


def _sc_aggregate(x, edges):
    mesh = plsc.VectorSubcoreMesh(core_axis_name="c", subcore_axis_name="s")

    @functools.partial(
        pl.kernel,
        mesh=mesh,
        out_type=jax.ShapeDtypeStruct((NC, N_NODES, HIDDEN), jnp.float32),
        scratch_types=[
            pltpu.VMEM_SHARED((N_NODES, HIDDEN), jnp.float32),
            pltpu.VMEM((EPW,), jnp.int32),
            pltpu.VMEM((EPW,), jnp.int32),
            pltpu.VMEM((NBUF, CHUNK, HIDDEN), jnp.float32),
            pltpu.SemaphoreType.DMA((NBUF,)),
            pltpu.SemaphoreType.DMA((NBUF,)),
            pltpu.SemaphoreType.DMA,
            pltpu.SemaphoreType.DMA,
        ],
    )
    def k(x_hbm, e_hbm, out_hbm, acc, src_v, dst_v, rows_v,
          gsem, ssem, isem, esem):
        c = lax.axis_index("c")
        s = lax.axis_index("s")
        wid = s * NC + c
        pltpu.async_copy(e_hbm.at[pl.ds(wid * EPW, EPW)], src_v, esem)
        pltpu.async_copy(e_hbm.at[pl.ds(N_EDGES + wid * EPW, EPW)],
                         dst_v, esem)
        @pl.when(jnp.logical_and(c == 0, s < NS - 1))
        def _():
            pltpu.async_copy(x_hbm.at[pl.ds(s * RPT, RPT)],
                             acc.at[pl.ds(s * RPT, RPT)], isem)
        @pl.when(jnp.logical_and(c == 0, s == NS - 1))
        def _():
            pltpu.async_copy(x_hbm.at[pl.ds((NS - 1) * RPT, RPT_LAST)],
                             acc.at[pl.ds((NS - 1) * RPT, RPT_LAST)], isem)
        @pl.when(c == 1)
        def _():
            z = rows_v.at[0]
            def zbody(r, carry):
                for q in range(HIDDEN // 16):
                    z[r, pl.ds(q * 16, 16)] = jnp.zeros((16,), jnp.float32)
                return carry
            lax.fori_loop(0, CHUNK, zbody, 0)
            @pl.when(s < NS - 1)
            def _():
                for kk in range(RPT // CHUNK):
                    pltpu.sync_copy(z, acc.at[pl.ds(s * RPT + kk * CHUNK,
                                                    CHUNK)])
                pltpu.sync_copy(
                    z.at[pl.ds(0, RPT % CHUNK)],
                    acc.at[pl.ds(s * RPT + (RPT // CHUNK) * CHUNK,
                                 RPT % CHUNK)])
            @pl.when(s == NS - 1)
            def _():
                for kk in range(RPT_LAST // CHUNK):
                    pltpu.sync_copy(
                        z, acc.at[pl.ds((NS - 1) * RPT + kk * CHUNK, CHUNK)])
                pltpu.sync_copy(
                    z.at[pl.ds(0, RPT_LAST % CHUNK)],
                    acc.at[pl.ds((NS - 1) * RPT + (RPT_LAST // CHUNK) * CHUNK,
                                 RPT_LAST % CHUNK)])
        pltpu.make_async_copy(e_hbm.at[pl.ds(wid * EPW, EPW)],
                              src_v, esem).wait()
        pltpu.make_async_copy(e_hbm.at[pl.ds(N_EDGES + wid * EPW, EPW)],
                              dst_v, esem).wait()

        GSUB = 2
        GPART = CHUNK // GSUB

        def start_gather(j, b):
            for i in range(GSUB):
                pltpu.async_copy(
                    x_hbm.at[src_v.at[pl.ds(j * CHUNK + i * GPART, GPART)]],
                    rows_v.at[b].at[pl.ds(i * GPART, GPART)], gsem.at[b])

        def wait_gather(j, b):
            for i in range(GSUB):
                pltpu.make_async_copy(
                    x_hbm.at[src_v.at[pl.ds(j * CHUNK + i * GPART, GPART)]],
                    rows_v.at[b].at[pl.ds(i * GPART, GPART)], gsem.at[b]).wait()

        def start_scatter(j, b):
            for i in range(CHUNK // 16):
                idx = dst_v[pl.ds(j * CHUNK + i * 16, 16)]
                pltpu.async_copy(rows_v.at[b].at[pl.ds(i * 16, 16)],
                                 acc.at[idx], ssem.at[b], add=True)

        def wait_scatter(j, b):
            for i in range(CHUNK // 16):
                idx = dst_v[pl.ds(j * CHUNK + i * 16, 16)]
                pltpu.make_async_copy(rows_v.at[b].at[pl.ds(i * 16, 16)],
                                      acc.at[idx], ssem.at[b]).wait()

        for b in range(LEAD + 1):
            start_gather(b, b)
        @pl.when(jnp.logical_and(c == 0, s < NS - 1))
        def _():
            pltpu.make_async_copy(x_hbm.at[pl.ds(s * RPT, RPT)],
                                  acc.at[pl.ds(s * RPT, RPT)], isem).wait()
        @pl.when(jnp.logical_and(c == 0, s == NS - 1))
        def _():
            pltpu.make_async_copy(
                x_hbm.at[pl.ds((NS - 1) * RPT, RPT_LAST)],
                acc.at[pl.ds((NS - 1) * RPT, RPT_LAST)], isem).wait()
        plsc.subcore_barrier()
        wait_gather(0, 0)
        start_scatter(0, 0)

        def body(g, carry):
            for b in range(NBUF):
                j = g * NBUF + b
                wait_scatter(j - NBUF, b)
                start_gather(j, b)
                jq = j - LEAD
                bq = (b + NBUF - LEAD) % NBUF
                wait_gather(jq, bq)
                start_scatter(jq, bq)
            return carry

        G = (NCHUNK - LEAD) // NBUF
        lax.fori_loop(1, G, body, 0)
        for j in range(G * NBUF, NCHUNK):
            b = j % NBUF
            wait_scatter(j - NBUF, b)
            start_gather(j, b)
            jq = j - LEAD
            wait_gather(jq, jq % NBUF)
            start_scatter(jq, jq % NBUF)
        for jq in range(NCHUNK, NCHUNK + LEAD):
            wait_gather(jq - LEAD, (jq - LEAD) % NBUF)
            start_scatter(jq - LEAD, (jq - LEAD) % NBUF)
        for jt in range(NCHUNK - NBUF, NCHUNK):
            wait_scatter(jt, jt % NBUF)
        plsc.subcore_barrier()
        @pl.when(s < NS - 1)
        def _():
            pltpu.sync_copy(acc.at[pl.ds(s * RPT, RPT)],
                            out_hbm.at[c].at[pl.ds(s * RPT, RPT)])
        @pl.when(s == NS - 1)
        def _():
            pltpu.sync_copy(acc.at[pl.ds((NS - 1) * RPT, RPT_LAST)],
                            out_hbm.at[c].at[pl.ds((NS - 1) * RPT, RPT_LAST)])

    return k(x, edges)


def _tc_dense(partials, W, b, gamma, beta):
    def body(p_ref, w_ref, b_ref, g_ref, be_ref, o_ref):
        h = p_ref[0] + p_ref[1]
        h = jnp.dot(h, w_ref[...], preferred_element_type=jnp.float32)
        h = h + b_ref[...]
        mean = jnp.mean(h, axis=0, keepdims=True)
        var = jnp.mean((h - mean) * (h - mean), axis=0, keepdims=True)
        h = (h - mean) * lax.rsqrt(var + 1e-5) * g_ref[...] + be_ref[...]
        o_ref[...] = jnp.maximum(h, 0.0)

    return pl.pallas_call(
        body,
        out_shape=jax.ShapeDtypeStruct((N_NODES, HIDDEN), jnp.float32),
    )(partials, W, b.reshape(1, HIDDEN), gamma.reshape(1, HIDDEN),
      beta.reshape(1, HIDDEN))


def kernel(x, edge_index, batch, W, b, gamma, beta):
    del batch
    edges = edge_index.astype(jnp.int32).reshape(2 * N_EDGES)
    partials = _sc_aggregate(x, edges)
    return _tc_dense(partials, W, b, gamma, beta)
